# Initial kernel scaffold; baseline (speedup 1.0000x reference)
import jax
import jax.numpy as jnp
from jax.experimental import pallas as pl


def kernel(x, table, W1, b1, W2, b2):
    raise NotImplementedError("write your pallas kernel here")


# speedup vs baseline: 30.2776x; 30.2776x over previous
import functools

import jax
import jax.numpy as jnp
from jax import lax
from jax.experimental import pallas as pl
from jax.experimental.pallas import tpu as pltpu
from jax.experimental.pallas import tpu_sc as plsc

B = 4096
LSEQ = 500
V = 10000
D = 100

NC = 2
NS = 16
L = 16
NW = NC * NS
BPW = B // NW

DP = 128
NCH = DP // L
LP = 512
GCH = 128
NG = LP // GCH
VP = V + L

INV_N = 1.0 / LSEQ


def _sc_pool(x3, table_p):
    mesh = plsc.VectorSubcoreMesh(
        core_axis_name="c", subcore_axis_name="s",
        num_cores=NC, num_subcores=NS)

    @functools.partial(
        pl.kernel,
        out_type=jax.ShapeDtypeStruct((B, DP), jnp.float32),
        mesh=mesh,
        scratch_types=[
            pltpu.VMEM((NG, GCH), jnp.int32),
            pltpu.VMEM((2, GCH, DP), jnp.float32),
            pltpu.VMEM((DP,), jnp.float32),
            pltpu.SemaphoreType.DMA,
            pltpu.SemaphoreType.DMA,
        ],
    )
    def k(x_hbm, tab_hbm, out_hbm, idx_v, rows_v, orow_v, sem0, sem1):
        wid = lax.axis_index("s") * NC + lax.axis_index("c")
        base = wid * BPW
        sems = (sem0, sem1)

        def row_body(j, _):
            b = base + j
            pltpu.sync_copy(x_hbm.at[b], idx_v)

            def start(kk):
                return pltpu.async_copy(
                    tab_hbm.at[idx_v.at[kk]], rows_v.at[kk % 2], sems[kk % 2])

            cp = start(0)
            acc = tuple(jnp.zeros((L,), jnp.float32) for _ in range(NCH))
            for kk in range(NG):
                nxt = start(kk + 1) if kk + 1 < NG else None
                cp.wait()

                def acc_body(r, a, _p=kk % 2):
                    return tuple(
                        a[c] + rows_v[_p, r, pl.ds(c * L, L)]
                        for c in range(NCH))

                acc = lax.fori_loop(0, GCH, acc_body, acc)
                cp = nxt

            for c in range(NCH):
                orow_v[pl.ds(c * L, L)] = acc[c]
            pltpu.sync_copy(orow_v, out_hbm.at[b])
            return 0

        lax.fori_loop(0, BPW, row_body, 0)

    return k(x3, table_p)


def _tc_mlp(psum, W1p, b1, W2, b2):
    BLK = 512

    def body(p_ref, w1_ref, b1_ref, w2_ref, b2_ref, o_ref):
        p = p_ref[...] * INV_N
        h = jnp.dot(p, w1_ref[...], preferred_element_type=jnp.float32)
        h = jnp.maximum(h + b1_ref[...], 0.0)
        z = jnp.dot(h, w2_ref[...], preferred_element_type=jnp.float32)
        z = z + b2_ref[...]
        m = jnp.max(z, axis=-1, keepdims=True)
        e = jnp.exp(z - m)
        o_ref[...] = e / jnp.sum(e, axis=-1, keepdims=True)

    return pl.pallas_call(
        body,
        grid=(B // BLK,),
        in_specs=[
            pl.BlockSpec((BLK, DP), lambda i: (i, 0)),
            pl.BlockSpec((DP, 160), lambda i: (0, 0)),
            pl.BlockSpec((1, 160), lambda i: (0, 0)),
            pl.BlockSpec((160, 1), lambda i: (0, 0)),
            pl.BlockSpec((1, 1), lambda i: (0, 0)),
        ],
        out_specs=pl.BlockSpec((BLK, 1), lambda i: (i, 0)),
        out_shape=jax.ShapeDtypeStruct((B, 1), jnp.float32),
    )(psum, W1p, b1.reshape(1, 160), W2, b2.reshape(1, 1))


def kernel(x, table, W1, b1, W2, b2):
    x = x.astype(jnp.int32)
    table_p = jnp.pad(table, ((0, VP - V), (0, DP - D)))
    pad = (V + (jnp.arange(LP - LSEQ, dtype=jnp.int32) % L))[None, :]
    xp = jnp.concatenate([x, jnp.broadcast_to(pad, (B, LP - LSEQ))], axis=1)
    x3 = xp.reshape(B, NG, GCH)

    psum = _sc_pool(x3, table_p)

    W1p = jnp.pad(W1, ((0, DP - D), (0, 0)))
    return _tc_mlp(psum, W1p, b1, W2, b2)
